# Initial kernel scaffold; baseline (speedup 1.0000x reference)
import jax
import jax.numpy as jnp
from jax.experimental import pallas as pl


def kernel(pred, target, length):
    raise NotImplementedError("write your pallas kernel here")


# speedup vs baseline: 2.0182x; 2.0182x over previous
import functools

import jax
import jax.numpy as jnp
from jax.experimental import pallas as pl
from jax.experimental.pallas import tpu as pltpu

_B, _T, _C = 8, 256, 32000
_SMOOTHING = 0.1
_CONFIDENCE = 1.0 - _SMOOTHING
_SMOOTH_VAL = _SMOOTHING / (_C - 1)
_IGNORE_INDEX = 0

_ROWS = _B * _T
_BLK = 64
_NBLK = _ROWS // _BLK


def _loss_body(pred_ref, tgt_ref, w_ref, denom_ref, out_ref):
    i = pl.program_id(0)
    p = pred_ref[...]
    m = jnp.max(p, axis=1, keepdims=True)
    s = jnp.sum(jnp.exp(p - m), axis=1, keepdims=True)
    lse = m + jnp.log(s)
    tot = jnp.sum(p, axis=1, keepdims=True)
    tgt = tgt_ref[0, 0, :]
    ids = jax.lax.broadcasted_iota(jnp.int32, p.shape, 1)
    pt = jnp.sum(jnp.where(ids == tgt[:, None], p, 0.0), axis=1, keepdims=True)
    w = w_ref[0, 0, :][:, None]
    tok = _SMOOTH_VAL * (tot - _C * lse) + (_CONFIDENCE - _SMOOTH_VAL) * (pt - lse)
    partial = -jnp.sum(tok * w)

    @pl.when(i == 0)
    def _init():
        out_ref[0, 0] = 0.0

    out_ref[0, 0] += partial

    @pl.when(i == pl.num_programs(0) - 1)
    def _fin():
        out_ref[0, 0] = out_ref[0, 0] / denom_ref[0, 0]


@jax.jit
def kernel(pred, target, length):
    pred2d = pred.reshape(_ROWS, _C)
    tflat = target.reshape(-1).astype(jnp.int32)
    tgt = jnp.concatenate([tflat[1:], jnp.zeros((1,), jnp.int32)])
    r = jnp.arange(_ROWS, dtype=jnp.int32)
    lim = (length - 1).astype(jnp.int32)[r // _T]
    valid = (r % _T) < lim
    ignored = valid & (tgt == _IGNORE_INDEX)
    w = (valid & ~ignored).astype(jnp.float32)
    denom = (jnp.sum(length - 1) - jnp.sum(ignored)).astype(jnp.float32)

    out = pl.pallas_call(
        _loss_body,
        grid=(_NBLK,),
        in_specs=[
            pl.BlockSpec((_BLK, _C), lambda i: (i, 0)),
            pl.BlockSpec((1, 1, _BLK), lambda i: (i, 0, 0)),
            pl.BlockSpec((1, 1, _BLK), lambda i: (i, 0, 0)),
            pl.BlockSpec(memory_space=pltpu.SMEM),
        ],
        out_specs=pl.BlockSpec(memory_space=pltpu.SMEM),
        out_shape=jax.ShapeDtypeStruct((1, 1), jnp.float32),
    )(
        pred2d,
        tgt.reshape(_NBLK, 1, _BLK),
        w.reshape(_NBLK, 1, _BLK),
        denom.reshape(1, 1),
    )
    return out[0, 0]


# speedup vs baseline: 2.2105x; 1.0953x over previous
import jax
import jax.numpy as jnp
from jax.experimental import pallas as pl
from jax.experimental.pallas import tpu as pltpu

_B, _T, _C = 8, 256, 32000
_SMOOTHING = 0.1
_CONFIDENCE = 1.0 - _SMOOTHING
_SMOOTH_VAL = _SMOOTHING / (_C - 1)
_IGNORE_INDEX = 0

_ROWS = _B * _T
_BLK = 32
_NBLK = _ROWS // _BLK


def _loss_body(bmap_ref, pred_ref, tgt_ref, w_ref, denom_ref, out_ref):
    i = pl.program_id(0)

    @pl.when(i == 0)
    def _init():
        out_ref[0, 0] = 0.0

    @pl.when(bmap_ref[i] == i)
    def _active():
        p = pred_ref[...]
        m = jnp.max(p, axis=1, keepdims=True)
        s = jnp.sum(jnp.exp(p - m), axis=1, keepdims=True)
        lse = m + jnp.log(s)
        tot = jnp.sum(p, axis=1, keepdims=True)
        tgt = tgt_ref[0, 0, :]
        ids = jax.lax.broadcasted_iota(jnp.int32, p.shape, 1)
        pt = jnp.sum(jnp.where(ids == tgt[:, None], p, 0.0),
                     axis=1, keepdims=True)
        w = w_ref[0, 0, :][:, None]
        tok = (_SMOOTH_VAL * (tot - _C * lse)
               + (_CONFIDENCE - _SMOOTH_VAL) * (pt - lse))
        out_ref[0, 0] += -jnp.sum(tok * w)

    @pl.when(i == pl.num_programs(0) - 1)
    def _fin():
        out_ref[0, 0] = out_ref[0, 0] / denom_ref[0]


@jax.jit
def kernel(pred, target, length):
    pred2d = pred.reshape(_ROWS, _C)
    tflat = target.reshape(-1).astype(jnp.int32)
    tgt = jnp.concatenate([tflat[1:], jnp.zeros((1,), jnp.int32)])
    r = jnp.arange(_ROWS, dtype=jnp.int32)
    lim = (length - 1).astype(jnp.int32)[r // _T]
    valid = (r % _T) < lim
    ignored = valid & (tgt == _IGNORE_INDEX)
    w = (valid & ~ignored).astype(jnp.float32)
    denom = (jnp.sum(length - 1) - jnp.sum(ignored)).astype(jnp.float32)

    blk = jnp.arange(_NBLK, dtype=jnp.int32)
    live = ((blk * _BLK) % _T) < lim[blk * _BLK]
    bmap = jax.lax.cummax(jnp.where(live, blk, -1))

    grid_spec = pltpu.PrefetchScalarGridSpec(
        num_scalar_prefetch=1,
        grid=(_NBLK,),
        in_specs=[
            pl.BlockSpec((_BLK, _C), lambda i, bmap: (bmap[i], 0)),
            pl.BlockSpec((1, 1, _BLK), lambda i, bmap: (i, 0, 0)),
            pl.BlockSpec((1, 1, _BLK), lambda i, bmap: (i, 0, 0)),
            pl.BlockSpec(memory_space=pltpu.SMEM),
        ],
        out_specs=pl.BlockSpec(memory_space=pltpu.SMEM),
    )
    out = pl.pallas_call(
        _loss_body,
        grid_spec=grid_spec,
        out_shape=jax.ShapeDtypeStruct((1, 1), jnp.float32),
    )(
        bmap,
        pred2d,
        tgt.reshape(_NBLK, 1, _BLK),
        w.reshape(_NBLK, 1, _BLK),
        denom.reshape(1),
    )
    return out[0, 0]


# speedup vs baseline: 2.3492x; 1.0628x over previous
import jax
import jax.numpy as jnp
from jax.experimental import pallas as pl
from jax.experimental.pallas import tpu as pltpu

_B, _T, _C = 8, 256, 32000
_SMOOTHING = 0.1
_CONFIDENCE = 1.0 - _SMOOTHING
_SMOOTH_VAL = _SMOOTHING / (_C - 1)
_IGNORE_INDEX = 0

_ROWS = _B * _T
_BLK = 32
_NBLK = _ROWS // _BLK


def _loss_body(bmap_ref, pred_ref, tgt_ref, w_ref, denom_ref, out_ref):
    i = pl.program_id(0)

    @pl.when(i == 0)
    def _init():
        out_ref[0, 0] = 0.0

    @pl.when(bmap_ref[i] == i)
    def _active():
        p = pred_ref[...]
        m = jnp.max(p, axis=1, keepdims=True)
        s = jnp.sum(jnp.exp(p - m), axis=1, keepdims=True)
        lse = m + jnp.log(s)
        tot = jnp.sum(p, axis=1, keepdims=True)
        tgt = tgt_ref[pl.ds(i, 1), :][0]
        ids = jax.lax.broadcasted_iota(jnp.int32, p.shape, 1)
        pt = jnp.sum(jnp.where(ids == tgt[:, None], p, 0.0),
                     axis=1, keepdims=True)
        w = w_ref[pl.ds(i, 1), :][0][:, None]
        tok = (_SMOOTH_VAL * (tot - _C * lse)
               + (_CONFIDENCE - _SMOOTH_VAL) * (pt - lse))
        out_ref[0, 0] += -jnp.sum(tok * w)

    @pl.when(i == pl.num_programs(0) - 1)
    def _fin():
        out_ref[0, 0] = out_ref[0, 0] / denom_ref[0]


@jax.jit
def kernel(pred, target, length):
    pred2d = pred.reshape(_ROWS, _C)
    tflat = target.reshape(-1).astype(jnp.int32)
    tgt = jnp.concatenate([tflat[1:], jnp.zeros((1,), jnp.int32)])
    r = jnp.arange(_ROWS, dtype=jnp.int32)
    lim = (length - 1).astype(jnp.int32)[r // _T]
    valid = (r % _T) < lim
    ignored = valid & (tgt == _IGNORE_INDEX)
    w = (valid & ~ignored).astype(jnp.float32)
    denom = (jnp.sum(length - 1) - jnp.sum(ignored)).astype(jnp.float32)

    blk = jnp.arange(_NBLK, dtype=jnp.int32)
    live = ((blk * _BLK) % _T) < lim[blk * _BLK]
    bmap = jax.lax.cummax(jnp.where(live, blk, -1))

    grid_spec = pltpu.PrefetchScalarGridSpec(
        num_scalar_prefetch=1,
        grid=(_NBLK,),
        in_specs=[
            pl.BlockSpec((_BLK, _C), lambda i, bmap: (bmap[i], 0)),
            pl.BlockSpec((_NBLK, _BLK), lambda i, bmap: (0, 0)),
            pl.BlockSpec((_NBLK, _BLK), lambda i, bmap: (0, 0)),
            pl.BlockSpec(memory_space=pltpu.SMEM),
        ],
        out_specs=pl.BlockSpec(memory_space=pltpu.SMEM),
    )
    out = pl.pallas_call(
        _loss_body,
        grid_spec=grid_spec,
        out_shape=jax.ShapeDtypeStruct((1, 1), jnp.float32),
    )(
        bmap,
        pred2d,
        tgt.reshape(_NBLK, _BLK),
        w.reshape(_NBLK, _BLK),
        denom.reshape(1),
    )
    return out[0, 0]


# speedup vs baseline: 2.6400x; 1.1238x over previous
import jax
import jax.numpy as jnp
from jax import lax
from jax.experimental import pallas as pl
from jax.experimental.pallas import tpu as pltpu

_B, _T, _C = 8, 256, 32000
_SMOOTHING = 0.1
_CONFIDENCE = 1.0 - _SMOOTHING
_SMOOTH_VAL = _SMOOTHING / (_C - 1)
_IGNORE_INDEX = 0

_ROWS = _B * _T
_BLK = 32
_NBLK = _ROWS // _BLK


def _loss_body(nlive_ref, bidx_ref, denom_ref, pred_ref, tgt_ref, w_ref,
               out_ref, buf_ref, sem):
    nlive = nlive_ref[0]

    def _copy(j):
        blk_id = bidx_ref[j]
        slot = lax.rem(j, 2)
        return pltpu.make_async_copy(
            pred_ref.at[pl.ds(blk_id * _BLK, _BLK), :],
            buf_ref.at[slot],
            sem.at[slot],
        )

    _copy(0).start()

    def _step(j, acc):
        @pl.when(j + 1 < nlive)
        def _prefetch():
            _copy(j + 1).start()

        _copy(j).wait()
        blk_id = bidx_ref[j]
        p = buf_ref[lax.rem(j, 2)]
        m = jnp.max(p, axis=1, keepdims=True)
        s = jnp.sum(jnp.exp(p - m), axis=1, keepdims=True)
        lse = m + jnp.log(s)
        tot = jnp.sum(p, axis=1, keepdims=True)
        tgt = tgt_ref[pl.ds(blk_id, 1), :][0]
        ids = jax.lax.broadcasted_iota(jnp.int32, p.shape, 1)
        pt = jnp.sum(jnp.where(ids == tgt[:, None], p, 0.0),
                     axis=1, keepdims=True)
        w = w_ref[pl.ds(blk_id, 1), :][0][:, None]
        tok = (_SMOOTH_VAL * (tot - _C * lse)
               + (_CONFIDENCE - _SMOOTH_VAL) * (pt - lse))
        return acc - jnp.sum(tok * w)

    acc = lax.fori_loop(0, nlive, _step, jnp.float32(0.0))
    out_ref[0, 0] = acc / denom_ref[0]


@jax.jit
def kernel(pred, target, length):
    pred2d = pred.reshape(_ROWS, _C)
    tflat = target.reshape(-1).astype(jnp.int32)
    tgt = jnp.concatenate([tflat[1:], jnp.zeros((1,), jnp.int32)])
    r = jnp.arange(_ROWS, dtype=jnp.int32)
    lim = (length - 1).astype(jnp.int32)[r // _T]
    valid = (r % _T) < lim
    ignored = valid & (tgt == _IGNORE_INDEX)
    w = (valid & ~ignored).astype(jnp.float32)
    denom = (jnp.sum(length - 1) - jnp.sum(ignored)).astype(jnp.float32)

    blk = jnp.arange(_NBLK, dtype=jnp.int32)
    live = ((blk * _BLK) % _T) < lim[blk * _BLK]
    nlive = jnp.sum(live.astype(jnp.int32))
    order = jnp.argsort(~live, stable=True).astype(jnp.int32)

    out = pl.pallas_call(
        _loss_body,
        in_specs=[
            pl.BlockSpec(memory_space=pltpu.SMEM),
            pl.BlockSpec(memory_space=pltpu.SMEM),
            pl.BlockSpec(memory_space=pltpu.SMEM),
            pl.BlockSpec(memory_space=pl.ANY),
            pl.BlockSpec(memory_space=pltpu.VMEM),
            pl.BlockSpec(memory_space=pltpu.VMEM),
        ],
        out_specs=pl.BlockSpec(memory_space=pltpu.SMEM),
        out_shape=jax.ShapeDtypeStruct((1, 1), jnp.float32),
        scratch_shapes=[
            pltpu.VMEM((2, _BLK, _C), jnp.float32),
            pltpu.SemaphoreType.DMA((2,)),
        ],
    )(
        nlive.reshape(1),
        order,
        denom.reshape(1),
        pred2d,
        tgt.reshape(_NBLK, _BLK),
        w.reshape(_NBLK, _BLK),
    )
    return out[0, 0]
